# baseline (speedup 1.0000x reference)
import functools

import jax
import jax.numpy as jnp
from jax import lax
from jax.experimental import pallas as pl
from jax.experimental.pallas import tpu as pltpu
from jax.experimental.pallas import tpu_sc as plsc

N_NODES = 10000
N_EDGES = 320000
D = 128
HD = D // 2

NC = 2
NS = 16
L = 16
NW = NC * NS
CH = 128
NCHUNK = 160
ROUNDS = 10
CPR = NCHUNK // ROUNDS
EPT = NCHUNK * CH
E_PAD = EPT * NS
NP_ = 10112
NBLK = NP_ // 128
ROWS_PT = NP_ // NS
NBUF = 4


def _sc_agg_body(with_cnt, *refs):
    if with_cnt:
        (x_hbm, src_hbm, dst_hbm, z_hbm, zflat_hbm, acc_out, cnt_out,
         src_v, dst_v, rows_v, cnt_v, x_sh, acc_sh,
         sem0, sem1, sem2, sem3) = refs
    else:
        (x_hbm, src_hbm, dst_hbm, z_hbm, acc_out,
         src_v, dst_v, rows_v, x_sh, acc_sh,
         sem0, sem1, sem2, sem3) = refs

    c = lax.axis_index("c")
    s = lax.axis_index("s")
    wid = c * NS + s
    row0 = s * ROWS_PT

    pltpu.sync_copy(x_hbm.at[pl.ds(row0, ROWS_PT), pl.ds(c * HD, HD)],
                    x_sh.at[pl.ds(row0, ROWS_PT)])
    pltpu.sync_copy(z_hbm, acc_sh.at[pl.ds(row0, ROWS_PT)])
    if with_cnt:
        pltpu.sync_copy(zflat_hbm, cnt_v)
    plsc.subcore_barrier()

    ones16 = jnp.ones((L,), jnp.float32)
    sems = (sem0, sem1, sem2, sem3)

    def start_gather(j, b):
        pltpu.async_copy(x_sh.at[src_v.at[j]], rows_v.at[b], sems[b])

    def wait_gather(j, b):
        pltpu.make_async_copy(x_sh.at[src_v.at[j]], rows_v.at[b],
                              sems[b]).wait()

    def scatter_chunk(j, b):
        pltpu.sync_copy(rows_v.at[b], acc_sh.at[dst_v.at[j]], add=True)

    def count_chunk(j):
        if not with_cnt:
            return
        for k in range(CH // L):
            d16 = dst_v[j, pl.ds(k * L, L)]
            plsc.addupdate_scatter(cnt_v, [d16], ones16)

    for r in range(ROUNDS):
        pltpu.sync_copy(src_hbm.at[s, pl.ds(r * CPR, CPR)], src_v)
        pltpu.sync_copy(dst_hbm.at[s, pl.ds(r * CPR, CPR)], dst_v)

        for b in range(NBUF):
            start_gather(b, b)
        for j in range(CPR):
            b = j % NBUF
            wait_gather(j, b)
            scatter_chunk(j, b)
            if j + NBUF < CPR:
                start_gather(j + NBUF, b)
            count_chunk(j)

    plsc.subcore_barrier()

    pltpu.sync_copy(acc_sh.at[pl.ds(row0, ROWS_PT)],
                    acc_out.at[c, pl.ds(row0, ROWS_PT)])
    if with_cnt:
        pltpu.sync_copy(cnt_v, cnt_out.at[wid])


def _make_sc_agg(with_cnt):
    mesh = plsc.VectorSubcoreMesh(core_axis_name="c", subcore_axis_name="s",
                                  num_cores=NC, num_subcores=NS)
    outs = [jax.ShapeDtypeStruct((NC, NP_, HD), jnp.float32)]
    scratch = [
        pltpu.VMEM((CPR, CH), jnp.int32),
        pltpu.VMEM((CPR, CH), jnp.int32),
        pltpu.VMEM((NBUF, CH, HD), jnp.float32),
    ]
    if with_cnt:
        outs.append(jax.ShapeDtypeStruct((NW, NP_), jnp.float32))
        scratch.append(pltpu.VMEM((NP_,), jnp.float32))
    scratch += [
        pltpu.VMEM_SHARED((NP_, HD), jnp.float32),
        pltpu.VMEM_SHARED((NP_, HD), jnp.float32),
        pltpu.SemaphoreType.DMA,
        pltpu.SemaphoreType.DMA,
        pltpu.SemaphoreType.DMA,
        pltpu.SemaphoreType.DMA,
    ]
    return pl.kernel(
        functools.partial(_sc_agg_body, with_cnt),
        out_type=tuple(outs) if with_cnt else outs[0],
        mesh=mesh,
        compiler_params=pltpu.CompilerParams(needs_layout_passes=False,
                                             use_tc_tiling_on_sc=False),
        scratch_types=scratch,
        name="sc_sage_agg" + ("_cnt" if with_cnt else ""),
    )


_sc_agg_cnt = _make_sc_agg(True)
_sc_agg_nocnt = _make_sc_agg(False)


def _dense_body(final, acc_ref, cnt_ref, x_ref, wl_ref, b_ref, wr_ref, *outs):
    acc = acc_ref[...]
    agg = jnp.concatenate([acc[0], acc[1]], axis=1)
    cnt = 0.5 * jnp.sum(cnt_ref[...], axis=0)
    cnt = jnp.maximum(cnt, 1.0)
    cm = jnp.broadcast_to(cnt[None, :], (128, 128))
    ir = lax.broadcasted_iota(jnp.int32, (128, 128), 0)
    ic = lax.broadcasted_iota(jnp.int32, (128, 128), 1)
    cnt_col = jnp.sum(jnp.where(ir == ic, cm, 0.0), axis=1, keepdims=True)
    mean = agg / cnt_col
    z = (jnp.dot(mean, wl_ref[...], preferred_element_type=jnp.float32)
         + b_ref[...]
         + jnp.dot(x_ref[...], wr_ref[...], preferred_element_type=jnp.float32))
    if final:
        m = jnp.max(z, axis=1, keepdims=True)
        e = jnp.exp(z - m)
        ssum = jnp.sum(e, axis=1, keepdims=True)
        outs[0][...] = z - m - jnp.log(ssum)
    else:
        outs[0][...] = jnp.maximum(z, 0.0)


def _make_dense(final):
    out_rows = N_NODES if final else NP_
    return pl.pallas_call(
        functools.partial(_dense_body, final),
        grid=(NBLK,),
        in_specs=[
            pl.BlockSpec((NC, 128, HD), lambda i: (0, i, 0)),
            pl.BlockSpec((NW, 128), lambda i: (0, i)),
            pl.BlockSpec((128, D), lambda i: (i, 0)),
            pl.BlockSpec((D, D), lambda i: (0, 0)),
            pl.BlockSpec((1, D), lambda i: (0, 0)),
            pl.BlockSpec((D, D), lambda i: (0, 0)),
        ],
        out_specs=pl.BlockSpec((128, D), lambda i: (i, 0)),
        out_shape=jax.ShapeDtypeStruct((out_rows, D), jnp.float32),
        name="tc_sage_dense" + ("2" if final else "1"),
    )


_dense1 = _make_dense(False)
_dense2 = _make_dense(True)


def kernel(x, edge_index, W1l, b1l, W1r, W2l, b2l, W2r):
    x_p = jnp.zeros((NP_, D), jnp.float32).at[:N_NODES].set(x)
    pad = E_PAD - N_EDGES
    src = jnp.concatenate(
        [edge_index[0], jnp.zeros((pad,), jnp.int32)]).reshape(NS, NCHUNK, CH)
    dst = jnp.concatenate(
        [edge_index[1], jnp.full((pad,), N_NODES, jnp.int32)]).reshape(NS, NCHUNK, CH)
    zrows = jnp.zeros((ROWS_PT, HD), jnp.float32)
    zflat = jnp.zeros((NP_,), jnp.float32)

    agg1, cnt = _sc_agg_cnt(x_p, src, dst, zrows, zflat)
    h = _dense1(agg1, cnt, x_p, W1l.T, b1l.reshape(1, D), W1r.T)
    agg2 = _sc_agg_nocnt(h, src, dst, zrows)
    return _dense2(agg2, cnt, h, W2l.T, b2l.reshape(1, D), W2r.T)


# speedup vs baseline: 1.0237x; 1.0237x over previous
import functools

import jax
import jax.numpy as jnp
from jax import lax
from jax.experimental import pallas as pl
from jax.experimental.pallas import tpu as pltpu
from jax.experimental.pallas import tpu_sc as plsc

N_NODES = 10000
N_EDGES = 320000
D = 128
HD = D // 2

NC = 2
NS = 16
L = 16
NW = NC * NS
CH = 128
NCHUNK = 160
ROUNDS = 10
CPR = NCHUNK // ROUNDS
EPT = NCHUNK * CH
E_PAD = EPT * NS
NP_ = 10112
NBLK = NP_ // 128
ROWS_PT = NP_ // NS
NBUF = 2


def _sc_agg_body(with_cnt, *refs):
    if with_cnt:
        (x_hbm, src_hbm, dst_hbm, z_hbm, zflat_hbm, acc_out, cnt_out,
         src_v, dst_v, rows_v, cnt_v, x_sh, acc_sh,
         sem0, sem1, sem2, sem3) = refs
    else:
        (x_hbm, src_hbm, dst_hbm, z_hbm, acc_out,
         src_v, dst_v, rows_v, x_sh, acc_sh,
         sem0, sem1, sem2, sem3) = refs

    c = lax.axis_index("c")
    s = lax.axis_index("s")
    wid = c * NS + s
    row0 = s * ROWS_PT

    pltpu.sync_copy(x_hbm.at[pl.ds(row0, ROWS_PT), pl.ds(c * HD, HD)],
                    x_sh.at[pl.ds(row0, ROWS_PT)])
    pltpu.sync_copy(z_hbm, acc_sh.at[pl.ds(row0, ROWS_PT)])
    if with_cnt:
        pltpu.sync_copy(zflat_hbm, cnt_v)
    plsc.subcore_barrier()

    ones16 = jnp.ones((L,), jnp.float32)
    sems = (sem0, sem1, sem2, sem3)

    def start_gather(j, b):
        pltpu.async_copy(x_sh.at[src_v.at[j]], rows_v.at[b], sems[b])

    def wait_gather(j, b):
        pltpu.make_async_copy(x_sh.at[src_v.at[j]], rows_v.at[b],
                              sems[b]).wait()

    def scatter_chunk(j, b):
        pltpu.sync_copy(rows_v.at[b], acc_sh.at[dst_v.at[j]], add=True)

    def count_chunk(j):
        if not with_cnt:
            return
        for k in range(CH // L):
            d16 = dst_v[j, pl.ds(k * L, L)]
            plsc.addupdate_scatter(cnt_v, [d16], ones16)

    for r in range(ROUNDS):
        pltpu.sync_copy(src_hbm.at[s, pl.ds(r * CPR, CPR)], src_v)
        pltpu.sync_copy(dst_hbm.at[s, pl.ds(r * CPR, CPR)], dst_v)

        for b in range(NBUF):
            start_gather(b, b)
        for j in range(CPR):
            b = j % NBUF
            wait_gather(j, b)
            scatter_chunk(j, b)
            if j + NBUF < CPR:
                start_gather(j + NBUF, b)
            count_chunk(j)

    plsc.subcore_barrier()

    pltpu.sync_copy(acc_sh.at[pl.ds(row0, ROWS_PT)],
                    acc_out.at[c, pl.ds(row0, ROWS_PT)])
    if with_cnt:
        pltpu.sync_copy(cnt_v, cnt_out.at[wid])


def _make_sc_agg(with_cnt):
    mesh = plsc.VectorSubcoreMesh(core_axis_name="c", subcore_axis_name="s",
                                  num_cores=NC, num_subcores=NS)
    outs = [jax.ShapeDtypeStruct((NC, NP_, HD), jnp.float32)]
    scratch = [
        pltpu.VMEM((CPR, CH), jnp.int32),
        pltpu.VMEM((CPR, CH), jnp.int32),
        pltpu.VMEM((NBUF, CH, HD), jnp.float32),
    ]
    if with_cnt:
        outs.append(jax.ShapeDtypeStruct((NW, NP_), jnp.float32))
        scratch.append(pltpu.VMEM((NP_,), jnp.float32))
    scratch += [
        pltpu.VMEM_SHARED((NP_, HD), jnp.float32),
        pltpu.VMEM_SHARED((NP_, HD), jnp.float32),
        pltpu.SemaphoreType.DMA,
        pltpu.SemaphoreType.DMA,
        pltpu.SemaphoreType.DMA,
        pltpu.SemaphoreType.DMA,
    ]
    return pl.kernel(
        functools.partial(_sc_agg_body, with_cnt),
        out_type=tuple(outs) if with_cnt else outs[0],
        mesh=mesh,
        compiler_params=pltpu.CompilerParams(needs_layout_passes=False,
                                             use_tc_tiling_on_sc=False),
        scratch_types=scratch,
        name="sc_sage_agg" + ("_cnt" if with_cnt else ""),
    )


_sc_agg_cnt = _make_sc_agg(True)
_sc_agg_nocnt = _make_sc_agg(False)


def _dense_body(final, acc_ref, cnt_ref, x_ref, wl_ref, b_ref, wr_ref, *outs):
    acc = acc_ref[...]
    agg = jnp.concatenate([acc[0], acc[1]], axis=1)
    cnt = 0.5 * jnp.sum(cnt_ref[...], axis=0)
    cnt = jnp.maximum(cnt, 1.0)
    cm = jnp.broadcast_to(cnt[None, :], (128, 128))
    ir = lax.broadcasted_iota(jnp.int32, (128, 128), 0)
    ic = lax.broadcasted_iota(jnp.int32, (128, 128), 1)
    cnt_col = jnp.sum(jnp.where(ir == ic, cm, 0.0), axis=1, keepdims=True)
    mean = agg / cnt_col
    z = (jnp.dot(mean, wl_ref[...], preferred_element_type=jnp.float32)
         + b_ref[...]
         + jnp.dot(x_ref[...], wr_ref[...], preferred_element_type=jnp.float32))
    if final:
        m = jnp.max(z, axis=1, keepdims=True)
        e = jnp.exp(z - m)
        ssum = jnp.sum(e, axis=1, keepdims=True)
        outs[0][...] = z - m - jnp.log(ssum)
    else:
        outs[0][...] = jnp.maximum(z, 0.0)


def _make_dense(final):
    out_rows = N_NODES if final else NP_
    return pl.pallas_call(
        functools.partial(_dense_body, final),
        grid=(NBLK,),
        in_specs=[
            pl.BlockSpec((NC, 128, HD), lambda i: (0, i, 0)),
            pl.BlockSpec((NW, 128), lambda i: (0, i)),
            pl.BlockSpec((128, D), lambda i: (i, 0)),
            pl.BlockSpec((D, D), lambda i: (0, 0)),
            pl.BlockSpec((1, D), lambda i: (0, 0)),
            pl.BlockSpec((D, D), lambda i: (0, 0)),
        ],
        out_specs=pl.BlockSpec((128, D), lambda i: (i, 0)),
        out_shape=jax.ShapeDtypeStruct((out_rows, D), jnp.float32),
        name="tc_sage_dense" + ("2" if final else "1"),
    )


_dense1 = _make_dense(False)
_dense2 = _make_dense(True)


def kernel(x, edge_index, W1l, b1l, W1r, W2l, b2l, W2r):
    x_p = jnp.zeros((NP_, D), jnp.float32).at[:N_NODES].set(x)
    pad = E_PAD - N_EDGES
    src = jnp.concatenate(
        [edge_index[0], jnp.zeros((pad,), jnp.int32)]).reshape(NS, NCHUNK, CH)
    dst = jnp.concatenate(
        [edge_index[1], jnp.full((pad,), N_NODES, jnp.int32)]).reshape(NS, NCHUNK, CH)
    zrows = jnp.zeros((ROWS_PT, HD), jnp.float32)
    zflat = jnp.zeros((NP_,), jnp.float32)

    agg1, cnt = _sc_agg_cnt(x_p, src, dst, zrows, zflat)
    h = _dense1(agg1, cnt, x_p, W1l.T, b1l.reshape(1, D), W1r.T)
    agg2 = _sc_agg_nocnt(h, src, dst, zrows)
    return _dense2(agg2, cnt, h, W2l.T, b2l.reshape(1, D), W2r.T)


# speedup vs baseline: 1.1250x; 1.0990x over previous
import functools

import jax
import jax.numpy as jnp
from jax import lax
from jax.experimental import pallas as pl
from jax.experimental.pallas import tpu as pltpu
from jax.experimental.pallas import tpu_sc as plsc

N_NODES = 10000
N_EDGES = 320000
D = 128
HD = D // 2

NC = 2
NS = 16
L = 16
NW = NC * NS
CH = 128
NCHUNK = 160
ROUNDS = 10
CPR = NCHUNK // ROUNDS
EPT = NCHUNK * CH
E_PAD = EPT * NS
NP_ = 10112
NBLK = NP_ // 128
BROW = 256
ROWS_PT = NP_ // NS
NBUF = 2


def _sc_agg_body(with_cnt, *refs):
    if with_cnt:
        (x_hbm, src_hbm, dst_hbm, z_hbm, zflat_hbm, acc_out, cnt_out,
         src_v, dst_v, rows_v, cnt_v, x_sh, acc_sh,
         sem0, sem1, sem2, sem3) = refs
    else:
        (x_hbm, src_hbm, dst_hbm, z_hbm, acc_out,
         src_v, dst_v, rows_v, x_sh, acc_sh,
         sem0, sem1, sem2, sem3) = refs

    c = lax.axis_index("c")
    s = lax.axis_index("s")
    wid = c * NS + s
    row0 = s * ROWS_PT

    pltpu.sync_copy(x_hbm.at[pl.ds(row0, ROWS_PT), pl.ds(c * HD, HD)],
                    x_sh.at[pl.ds(row0, ROWS_PT)])
    pltpu.sync_copy(z_hbm, acc_sh.at[pl.ds(row0, ROWS_PT)])
    if with_cnt:
        pltpu.sync_copy(zflat_hbm, cnt_v)
    plsc.subcore_barrier()

    ones16 = jnp.ones((L,), jnp.float32)
    sems = (sem0, sem1, sem2, sem3)

    def start_gather(j, b):
        pltpu.async_copy(x_sh.at[src_v.at[j]], rows_v.at[b], sems[b])

    def wait_gather(j, b):
        pltpu.make_async_copy(x_sh.at[src_v.at[j]], rows_v.at[b],
                              sems[b]).wait()

    def scatter_chunk(j, b):
        pltpu.sync_copy(rows_v.at[b], acc_sh.at[dst_v.at[j]], add=True)

    def count_chunk(j):
        if not with_cnt:
            return
        for k in range(CH // L):
            d16 = dst_v[j, pl.ds(k * L, L)]
            plsc.addupdate_scatter(cnt_v, [d16], ones16)

    for r in range(ROUNDS):
        pltpu.sync_copy(src_hbm.at[s, pl.ds(r * CPR, CPR)], src_v)
        pltpu.sync_copy(dst_hbm.at[s, pl.ds(r * CPR, CPR)], dst_v)

        for b in range(NBUF):
            start_gather(b, b)
        for j in range(CPR):
            b = j % NBUF
            wait_gather(j, b)
            scatter_chunk(j, b)
            if j + NBUF < CPR:
                start_gather(j + NBUF, b)
            count_chunk(j)

    plsc.subcore_barrier()

    pltpu.sync_copy(acc_sh.at[pl.ds(row0, ROWS_PT)],
                    acc_out.at[c, pl.ds(row0, ROWS_PT)])
    if with_cnt:
        pltpu.sync_copy(cnt_v, cnt_out.at[wid])


def _make_sc_agg(with_cnt):
    mesh = plsc.VectorSubcoreMesh(core_axis_name="c", subcore_axis_name="s",
                                  num_cores=NC, num_subcores=NS)
    outs = [jax.ShapeDtypeStruct((NC, NP_, HD), jnp.float32)]
    scratch = [
        pltpu.VMEM((CPR, CH), jnp.int32),
        pltpu.VMEM((CPR, CH), jnp.int32),
        pltpu.VMEM((NBUF, CH, HD), jnp.float32),
    ]
    if with_cnt:
        outs.append(jax.ShapeDtypeStruct((NW, NP_), jnp.float32))
        scratch.append(pltpu.VMEM((NP_,), jnp.float32))
    scratch += [
        pltpu.VMEM_SHARED((NP_, HD), jnp.float32),
        pltpu.VMEM_SHARED((NP_, HD), jnp.float32),
        pltpu.SemaphoreType.DMA,
        pltpu.SemaphoreType.DMA,
        pltpu.SemaphoreType.DMA,
        pltpu.SemaphoreType.DMA,
    ]
    return pl.kernel(
        functools.partial(_sc_agg_body, with_cnt),
        out_type=tuple(outs) if with_cnt else outs[0],
        mesh=mesh,
        compiler_params=pltpu.CompilerParams(needs_layout_passes=False,
                                             use_tc_tiling_on_sc=False),
        scratch_types=scratch,
        name="sc_sage_agg" + ("_cnt" if with_cnt else ""),
    )


_sc_agg_cnt = _make_sc_agg(True)
_sc_agg_nocnt = _make_sc_agg(False)


def _dense_body(final, acc_ref, cnt_ref, x_ref, wl_ref, b_ref, wr_ref, *outs):
    acc = acc_ref[...]
    agg = jnp.concatenate([acc[0], acc[1]], axis=1)
    cnt = 0.5 * jnp.sum(cnt_ref[...], axis=0)
    cnt = jnp.maximum(cnt, 1.0)
    cnt2 = cnt.reshape(BROW // 128, 128)
    ir = lax.broadcasted_iota(jnp.int32, (128, 128), 0)
    ic = lax.broadcasted_iota(jnp.int32, (128, 128), 1)
    cols = [jnp.sum(jnp.where(ir == ic,
                              jnp.broadcast_to(cnt2[g][None, :], (128, 128)),
                              0.0), axis=1, keepdims=True)
            for g in range(BROW // 128)]
    cnt_col = jnp.concatenate(cols, axis=0)
    mean = agg / cnt_col
    z = (jnp.dot(mean, wl_ref[...], preferred_element_type=jnp.float32)
         + b_ref[...]
         + jnp.dot(x_ref[...], wr_ref[...], preferred_element_type=jnp.float32))
    if final:
        m = jnp.max(z, axis=1, keepdims=True)
        e = jnp.exp(z - m)
        ssum = jnp.sum(e, axis=1, keepdims=True)
        outs[0][...] = z - m - jnp.log(ssum)
    else:
        outs[0][...] = jnp.maximum(z, 0.0)


def _make_dense(final):
    out_rows = N_NODES if final else NP_
    return pl.pallas_call(
        functools.partial(_dense_body, final),
        grid=(pl.cdiv(NP_, BROW),),
        in_specs=[
            pl.BlockSpec((NC, BROW, HD), lambda i: (0, i, 0)),
            pl.BlockSpec((NW, BROW), lambda i: (0, i)),
            pl.BlockSpec((BROW, D), lambda i: (i, 0)),
            pl.BlockSpec((D, D), lambda i: (0, 0)),
            pl.BlockSpec((1, D), lambda i: (0, 0)),
            pl.BlockSpec((D, D), lambda i: (0, 0)),
        ],
        out_specs=pl.BlockSpec((BROW, D), lambda i: (i, 0)),
        out_shape=jax.ShapeDtypeStruct((out_rows, D), jnp.float32),
        name="tc_sage_dense" + ("2" if final else "1"),
    )


_dense1 = _make_dense(False)
_dense2 = _make_dense(True)


def kernel(x, edge_index, W1l, b1l, W1r, W2l, b2l, W2r):
    x_p = jnp.zeros((NP_, D), jnp.float32).at[:N_NODES].set(x)
    pad = E_PAD - N_EDGES
    src = jnp.concatenate(
        [edge_index[0], jnp.zeros((pad,), jnp.int32)]).reshape(NS, NCHUNK, CH)
    dst = jnp.concatenate(
        [edge_index[1], jnp.full((pad,), N_NODES, jnp.int32)]).reshape(NS, NCHUNK, CH)
    zrows = jnp.zeros((ROWS_PT, HD), jnp.float32)
    zflat = jnp.zeros((NP_,), jnp.float32)

    agg1, cnt = _sc_agg_cnt(x_p, src, dst, zrows, zflat)
    h = _dense1(agg1, cnt, x_p, W1l.T, b1l.reshape(1, D), W1r.T)
    agg2 = _sc_agg_nocnt(h, src, dst, zrows)
    return _dense2(agg2, cnt, h, W2l.T, b2l.reshape(1, D), W2r.T)


# speedup vs baseline: 1.1845x; 1.0529x over previous
import functools

import jax
import jax.numpy as jnp
from jax import lax
from jax.experimental import pallas as pl
from jax.experimental.pallas import tpu as pltpu
from jax.experimental.pallas import tpu_sc as plsc

N_NODES = 10000
N_EDGES = 320000
D = 128
HD = D // 2

NC = 2
NS = 16
L = 16
NW = NC * NS
CH = 128
NCHUNK = 160
ROUNDS = 10
CPR = NCHUNK // ROUNDS
EPT = NCHUNK * CH
E_PAD = EPT * NS
NP_ = 10112
NBLK = NP_ // 128
BROW = 512
ROWS_PT = NP_ // NS
NBUF = 2


def _sc_agg_body(with_cnt, *refs):
    if with_cnt:
        (x_hbm, src_hbm, dst_hbm, z_hbm, zflat_hbm, acc_out, cnt_out,
         src_v, dst_v, rows_v, cnt_v, x_sh, acc_sh,
         sem0, sem1, sem2, sem3) = refs
    else:
        (x_hbm, src_hbm, dst_hbm, z_hbm, acc_out,
         src_v, dst_v, rows_v, x_sh, acc_sh,
         sem0, sem1, sem2, sem3) = refs

    c = lax.axis_index("c")
    s = lax.axis_index("s")
    wid = c * NS + s
    row0 = s * ROWS_PT

    pltpu.sync_copy(x_hbm.at[pl.ds(row0, ROWS_PT), pl.ds(c * HD, HD)],
                    x_sh.at[pl.ds(row0, ROWS_PT)])
    pltpu.sync_copy(z_hbm, acc_sh.at[pl.ds(row0, ROWS_PT)])
    if with_cnt:
        pltpu.sync_copy(zflat_hbm, cnt_v)
    plsc.subcore_barrier()

    ones16 = jnp.ones((L,), jnp.float32)
    sems = (sem0, sem1, sem2, sem3)

    def start_gather(j, b):
        pltpu.async_copy(x_sh.at[src_v.at[j]], rows_v.at[b], sems[b])

    def wait_gather(j, b):
        pltpu.make_async_copy(x_sh.at[src_v.at[j]], rows_v.at[b],
                              sems[b]).wait()

    def scatter_chunk(j, b):
        pltpu.sync_copy(rows_v.at[b], acc_sh.at[dst_v.at[j]], add=True)

    def count_chunk(j):
        if not with_cnt:
            return
        for k in range(CH // L):
            d16 = dst_v[j, pl.ds(k * L, L)]
            plsc.addupdate_scatter(cnt_v, [d16], ones16)

    for r in range(ROUNDS):
        pltpu.sync_copy(src_hbm.at[s, pl.ds(r * CPR, CPR)], src_v)
        pltpu.sync_copy(dst_hbm.at[s, pl.ds(r * CPR, CPR)], dst_v)

        for b in range(NBUF):
            start_gather(b, b)
        for j in range(CPR):
            b = j % NBUF
            wait_gather(j, b)
            scatter_chunk(j, b)
            if j + NBUF < CPR:
                start_gather(j + NBUF, b)
            count_chunk(j)

    plsc.subcore_barrier()

    pltpu.sync_copy(acc_sh.at[pl.ds(row0, ROWS_PT)],
                    acc_out.at[c, pl.ds(row0, ROWS_PT)])
    if with_cnt:
        pltpu.sync_copy(cnt_v, cnt_out.at[wid])


def _make_sc_agg(with_cnt):
    mesh = plsc.VectorSubcoreMesh(core_axis_name="c", subcore_axis_name="s",
                                  num_cores=NC, num_subcores=NS)
    outs = [jax.ShapeDtypeStruct((NC, NP_, HD), jnp.float32)]
    scratch = [
        pltpu.VMEM((CPR, CH), jnp.int32),
        pltpu.VMEM((CPR, CH), jnp.int32),
        pltpu.VMEM((NBUF, CH, HD), jnp.float32),
    ]
    if with_cnt:
        outs.append(jax.ShapeDtypeStruct((NW, NP_), jnp.float32))
        scratch.append(pltpu.VMEM((NP_,), jnp.float32))
    scratch += [
        pltpu.VMEM_SHARED((NP_, HD), jnp.float32),
        pltpu.VMEM_SHARED((NP_, HD), jnp.float32),
        pltpu.SemaphoreType.DMA,
        pltpu.SemaphoreType.DMA,
        pltpu.SemaphoreType.DMA,
        pltpu.SemaphoreType.DMA,
    ]
    return pl.kernel(
        functools.partial(_sc_agg_body, with_cnt),
        out_type=tuple(outs) if with_cnt else outs[0],
        mesh=mesh,
        compiler_params=pltpu.CompilerParams(needs_layout_passes=False,
                                             use_tc_tiling_on_sc=False),
        scratch_types=scratch,
        name="sc_sage_agg" + ("_cnt" if with_cnt else ""),
    )


_sc_agg_cnt = _make_sc_agg(True)
_sc_agg_nocnt = _make_sc_agg(False)


def _dense_body(final, acc_ref, cnt_ref, x_ref, wl_ref, b_ref, wr_ref, *outs):
    acc = acc_ref[...]
    agg = jnp.concatenate([acc[0], acc[1]], axis=1)
    cnt = 0.5 * jnp.sum(cnt_ref[...], axis=0)
    cnt = jnp.maximum(cnt, 1.0)
    cnt2 = cnt.reshape(BROW // 128, 128)
    ir = lax.broadcasted_iota(jnp.int32, (128, 128), 0)
    ic = lax.broadcasted_iota(jnp.int32, (128, 128), 1)
    cols = [jnp.sum(jnp.where(ir == ic,
                              jnp.broadcast_to(cnt2[g][None, :], (128, 128)),
                              0.0), axis=1, keepdims=True)
            for g in range(BROW // 128)]
    cnt_col = jnp.concatenate(cols, axis=0)
    mean = agg / cnt_col
    z = (jnp.dot(mean, wl_ref[...], preferred_element_type=jnp.float32)
         + b_ref[...]
         + jnp.dot(x_ref[...], wr_ref[...], preferred_element_type=jnp.float32))
    if final:
        m = jnp.max(z, axis=1, keepdims=True)
        e = jnp.exp(z - m)
        ssum = jnp.sum(e, axis=1, keepdims=True)
        outs[0][...] = z - m - jnp.log(ssum)
    else:
        outs[0][...] = jnp.maximum(z, 0.0)


def _make_dense(final):
    out_rows = N_NODES if final else NP_
    return pl.pallas_call(
        functools.partial(_dense_body, final),
        grid=(pl.cdiv(NP_, BROW),),
        in_specs=[
            pl.BlockSpec((NC, BROW, HD), lambda i: (0, i, 0)),
            pl.BlockSpec((NW, BROW), lambda i: (0, i)),
            pl.BlockSpec((BROW, D), lambda i: (i, 0)),
            pl.BlockSpec((D, D), lambda i: (0, 0)),
            pl.BlockSpec((1, D), lambda i: (0, 0)),
            pl.BlockSpec((D, D), lambda i: (0, 0)),
        ],
        out_specs=pl.BlockSpec((BROW, D), lambda i: (i, 0)),
        out_shape=jax.ShapeDtypeStruct((out_rows, D), jnp.float32),
        name="tc_sage_dense" + ("2" if final else "1"),
    )


_dense1 = _make_dense(False)
_dense2 = _make_dense(True)


def kernel(x, edge_index, W1l, b1l, W1r, W2l, b2l, W2r):
    x_p = jnp.zeros((NP_, D), jnp.float32).at[:N_NODES].set(x)
    pad = E_PAD - N_EDGES
    src = jnp.concatenate(
        [edge_index[0], jnp.zeros((pad,), jnp.int32)]).reshape(NS, NCHUNK, CH)
    dst = jnp.concatenate(
        [edge_index[1], jnp.full((pad,), N_NODES, jnp.int32)]).reshape(NS, NCHUNK, CH)
    zrows = jnp.zeros((ROWS_PT, HD), jnp.float32)
    zflat = jnp.zeros((NP_,), jnp.float32)

    agg1, cnt = _sc_agg_cnt(x_p, src, dst, zrows, zflat)
    h = _dense1(agg1, cnt, x_p, W1l.T, b1l.reshape(1, D), W1r.T)
    agg2 = _sc_agg_nocnt(h, src, dst, zrows)
    return _dense2(agg2, cnt, h, W2l.T, b2l.reshape(1, D), W2r.T)


# speedup vs baseline: 1.2126x; 1.0237x over previous
import functools

import jax
import jax.numpy as jnp
from jax import lax
from jax.experimental import pallas as pl
from jax.experimental.pallas import tpu as pltpu
from jax.experimental.pallas import tpu_sc as plsc

N_NODES = 10000
N_EDGES = 320000
D = 128
HD = D // 2

NC = 2
NS = 16
L = 16
NW = NC * NS
CH = 128
NCHUNK = 160
ROUNDS = 10
CPR = NCHUNK // ROUNDS
EPT = NCHUNK * CH
E_PAD = EPT * NS
NP_ = 10112
NBLK = NP_ // 128
BROW = 1024
ROWS_PT = NP_ // NS
NBUF = 2


def _sc_agg_body(with_cnt, *refs):
    if with_cnt:
        (x_hbm, src_hbm, dst_hbm, z_hbm, zflat_hbm, acc_out, cnt_out,
         src_v, dst_v, rows_v, cnt_v, x_sh, acc_sh,
         sem0, sem1, sem2, sem3) = refs
    else:
        (x_hbm, src_hbm, dst_hbm, z_hbm, acc_out,
         src_v, dst_v, rows_v, x_sh, acc_sh,
         sem0, sem1, sem2, sem3) = refs

    c = lax.axis_index("c")
    s = lax.axis_index("s")
    wid = c * NS + s
    row0 = s * ROWS_PT

    pltpu.sync_copy(x_hbm.at[pl.ds(row0, ROWS_PT), pl.ds(c * HD, HD)],
                    x_sh.at[pl.ds(row0, ROWS_PT)])
    pltpu.sync_copy(z_hbm, acc_sh.at[pl.ds(row0, ROWS_PT)])
    if with_cnt:
        pltpu.sync_copy(zflat_hbm, cnt_v)
    plsc.subcore_barrier()

    ones16 = jnp.ones((L,), jnp.float32)
    sems = (sem0, sem1, sem2, sem3)

    def start_gather(j, b):
        pltpu.async_copy(x_sh.at[src_v.at[j]], rows_v.at[b], sems[b])

    def wait_gather(j, b):
        pltpu.make_async_copy(x_sh.at[src_v.at[j]], rows_v.at[b],
                              sems[b]).wait()

    def scatter_chunk(j, b):
        pltpu.sync_copy(rows_v.at[b], acc_sh.at[dst_v.at[j]], add=True)

    def count_chunk(j):
        if not with_cnt:
            return
        for k in range(CH // L):
            d16 = dst_v[j, pl.ds(k * L, L)]
            plsc.addupdate_scatter(cnt_v, [d16], ones16)

    for r in range(ROUNDS):
        pltpu.sync_copy(src_hbm.at[s, pl.ds(r * CPR, CPR)], src_v)
        pltpu.sync_copy(dst_hbm.at[s, pl.ds(r * CPR, CPR)], dst_v)

        for b in range(NBUF):
            start_gather(b, b)
        for j in range(CPR):
            b = j % NBUF
            wait_gather(j, b)
            scatter_chunk(j, b)
            if j + NBUF < CPR:
                start_gather(j + NBUF, b)
            count_chunk(j)

    plsc.subcore_barrier()

    pltpu.sync_copy(acc_sh.at[pl.ds(row0, ROWS_PT)],
                    acc_out.at[c, pl.ds(row0, ROWS_PT)])
    if with_cnt:
        pltpu.sync_copy(cnt_v, cnt_out.at[wid])


def _make_sc_agg(with_cnt):
    mesh = plsc.VectorSubcoreMesh(core_axis_name="c", subcore_axis_name="s",
                                  num_cores=NC, num_subcores=NS)
    outs = [jax.ShapeDtypeStruct((NC, NP_, HD), jnp.float32)]
    scratch = [
        pltpu.VMEM((CPR, CH), jnp.int32),
        pltpu.VMEM((CPR, CH), jnp.int32),
        pltpu.VMEM((NBUF, CH, HD), jnp.float32),
    ]
    if with_cnt:
        outs.append(jax.ShapeDtypeStruct((NW, NP_), jnp.float32))
        scratch.append(pltpu.VMEM((NP_,), jnp.float32))
    scratch += [
        pltpu.VMEM_SHARED((NP_, HD), jnp.float32),
        pltpu.VMEM_SHARED((NP_, HD), jnp.float32),
        pltpu.SemaphoreType.DMA,
        pltpu.SemaphoreType.DMA,
        pltpu.SemaphoreType.DMA,
        pltpu.SemaphoreType.DMA,
    ]
    return pl.kernel(
        functools.partial(_sc_agg_body, with_cnt),
        out_type=tuple(outs) if with_cnt else outs[0],
        mesh=mesh,
        compiler_params=pltpu.CompilerParams(needs_layout_passes=False,
                                             use_tc_tiling_on_sc=False),
        scratch_types=scratch,
        name="sc_sage_agg" + ("_cnt" if with_cnt else ""),
    )


_sc_agg_cnt = _make_sc_agg(True)
_sc_agg_nocnt = _make_sc_agg(False)


def _dense_body(final, acc_ref, cnt_ref, x_ref, wl_ref, b_ref, wr_ref, *outs):
    acc = acc_ref[...]
    agg = jnp.concatenate([acc[0], acc[1]], axis=1)
    cnt = 0.5 * jnp.sum(cnt_ref[...], axis=0)
    cnt = jnp.maximum(cnt, 1.0)
    cnt2 = cnt.reshape(BROW // 128, 128)
    ir = lax.broadcasted_iota(jnp.int32, (128, 128), 0)
    ic = lax.broadcasted_iota(jnp.int32, (128, 128), 1)
    cols = [jnp.sum(jnp.where(ir == ic,
                              jnp.broadcast_to(cnt2[g][None, :], (128, 128)),
                              0.0), axis=1, keepdims=True)
            for g in range(BROW // 128)]
    cnt_col = jnp.concatenate(cols, axis=0)
    mean = agg / cnt_col
    z = (jnp.dot(mean, wl_ref[...], preferred_element_type=jnp.float32)
         + b_ref[...]
         + jnp.dot(x_ref[...], wr_ref[...], preferred_element_type=jnp.float32))
    if final:
        m = jnp.max(z, axis=1, keepdims=True)
        e = jnp.exp(z - m)
        ssum = jnp.sum(e, axis=1, keepdims=True)
        outs[0][...] = z - m - jnp.log(ssum)
    else:
        outs[0][...] = jnp.maximum(z, 0.0)


def _make_dense(final):
    out_rows = N_NODES if final else NP_
    return pl.pallas_call(
        functools.partial(_dense_body, final),
        grid=(pl.cdiv(NP_, BROW),),
        in_specs=[
            pl.BlockSpec((NC, BROW, HD), lambda i: (0, i, 0)),
            pl.BlockSpec((NW, BROW), lambda i: (0, i)),
            pl.BlockSpec((BROW, D), lambda i: (i, 0)),
            pl.BlockSpec((D, D), lambda i: (0, 0)),
            pl.BlockSpec((1, D), lambda i: (0, 0)),
            pl.BlockSpec((D, D), lambda i: (0, 0)),
        ],
        out_specs=pl.BlockSpec((BROW, D), lambda i: (i, 0)),
        out_shape=jax.ShapeDtypeStruct((out_rows, D), jnp.float32),
        name="tc_sage_dense" + ("2" if final else "1"),
    )


_dense1 = _make_dense(False)
_dense2 = _make_dense(True)


def kernel(x, edge_index, W1l, b1l, W1r, W2l, b2l, W2r):
    x_p = jnp.zeros((NP_, D), jnp.float32).at[:N_NODES].set(x)
    pad = E_PAD - N_EDGES
    src = jnp.concatenate(
        [edge_index[0], jnp.zeros((pad,), jnp.int32)]).reshape(NS, NCHUNK, CH)
    dst = jnp.concatenate(
        [edge_index[1], jnp.full((pad,), N_NODES, jnp.int32)]).reshape(NS, NCHUNK, CH)
    zrows = jnp.zeros((ROWS_PT, HD), jnp.float32)
    zflat = jnp.zeros((NP_,), jnp.float32)

    agg1, cnt = _sc_agg_cnt(x_p, src, dst, zrows, zflat)
    h = _dense1(agg1, cnt, x_p, W1l.T, b1l.reshape(1, D), W1r.T)
    agg2 = _sc_agg_nocnt(h, src, dst, zrows)
    return _dense2(agg2, cnt, h, W2l.T, b2l.reshape(1, D), W2r.T)
